# baseline (speedup 1.0000x reference)
import functools

import jax
import jax.numpy as jnp
from jax import lax
from jax.experimental import pallas as pl
from jax.experimental.pallas import tpu as pltpu
from jax.experimental.pallas import tpu_sc as plsc

N = 10000
E = 320000
D_IN = 128
D_HID = 16
D_OUT = 40

NC = 2
NS = 16
NW = NC * NS
B = 128
NBAT = E // B
NB = NBAT // NW
NXTRA = NBAT - NB * NW
NBUF1 = 13
PIPE1 = 6
NBUF = 6
PIPE = 3
NPAD = 10112
RPT = NPAD // NS


def _seg_sum_pipeline(eidx, tab_sh, acc, src_v, dst_v, rows, srcx, dstx,
                      gsem, ssem, wid, nbuf, pipe):
    pltpu.sync_copy(eidx.at[0, pl.ds(wid * NB, NB)], src_v)
    pltpu.sync_copy(eidx.at[1, pl.ds(wid * NB, NB)], dst_v)

    for b in range(pipe):
        pltpu.async_copy(tab_sh.at[src_v.at[b]], rows[b], gsem[b])

    def group(g, _):
        j0 = g * nbuf
        for b in range(nbuf):
            j = j0 + b
            pltpu.make_async_copy(
                tab_sh.at[src_v.at[j]], rows[b], gsem[b]).wait()
            pltpu.async_copy(rows[b], acc.at[dst_v.at[j]], ssem[b],
                             add=True)
            b2 = (b + pipe) % nbuf
            jn = j + pipe
            prev = jn - nbuf

            @pl.when(jn < NB)
            def _():
                @pl.when(prev >= 0)
                def _():
                    pltpu.make_async_copy(
                        rows[b2], acc.at[dst_v.at[prev]], ssem[b2]).wait()
                pltpu.async_copy(tab_sh.at[src_v.at[jn]], rows[b2],
                                 gsem[b2])
        return 0

    lax.fori_loop(0, NB // nbuf, group, 0)
    for b in range(nbuf):
        pltpu.make_async_copy(
            rows[b], acc.at[dst_v.at[NB - nbuf + b]], ssem[b]).wait()

    @pl.when(wid < NXTRA)
    def _():
        pltpu.sync_copy(eidx.at[0, pl.ds(NB * NW + wid, 1)], srcx)
        pltpu.sync_copy(eidx.at[1, pl.ds(NB * NW + wid, 1)], dstx)
        pltpu.async_copy(tab_sh.at[srcx.at[0]], rows[0], gsem[0]).wait()
        pltpu.sync_copy(rows[0], acc.at[dstx.at[0]], add=True)


_MESH = plsc.VectorSubcoreMesh(core_axis_name="c", subcore_axis_name="s")
_SC_PARAMS = pltpu.CompilerParams(use_tc_tiling_on_sc=False)


@functools.partial(
    pl.kernel,
    out_type=jax.ShapeDtypeStruct((NC, NPAD, 32), jnp.float32),
    mesh=_MESH,
    scratch_types=[
        pltpu.VMEM((NB, B), jnp.int32),
        pltpu.VMEM((NB, B), jnp.int32),
        [pltpu.VMEM((B, 32), jnp.float32)] * NBUF1,
        pltpu.VMEM((1, B), jnp.int32),
        pltpu.VMEM((1, B), jnp.int32),
        pltpu.VMEM_SHARED((NPAD, 32), jnp.float32),
        pltpu.VMEM_SHARED((NPAD, 32), jnp.float32),
        [pltpu.SemaphoreType.DMA] * NBUF1,
        [pltpu.SemaphoreType.DMA] * NBUF1,
    ],
    compiler_params=_SC_PARAMS,
)
def _sc_layer1(t1x, eidx, out, src_v, dst_v, rows, srcx, dstx,
               acc, tab_sh, gsem, ssem):
    c = lax.axis_index("c")
    s = lax.axis_index("s")
    wid = c * NS + s
    r0 = s * RPT

    stage = pltpu.async_copy(t1x.at[pl.ds(r0, RPT), pl.ds(0, 32)],
                             tab_sh.at[pl.ds(r0, RPT)], gsem[0])

    def zero_body(i, _):
        for j in range(2):
            rows[0][i, pl.ds(j * 16, 16)] = jnp.zeros((16,), jnp.float32)
        return 0

    lax.fori_loop(0, B, zero_body, 0)
    nfull = RPT // B
    for k in range(nfull):
        pltpu.async_copy(rows[0], acc.at[pl.ds(r0 + k * B, B)],
                         ssem[k])
    rem = RPT - nfull * B
    pltpu.async_copy(rows[0].at[pl.ds(0, rem)],
                     acc.at[pl.ds(r0 + nfull * B, rem)], ssem[nfull])
    for k in range(nfull):
        pltpu.make_async_copy(rows[0], acc.at[pl.ds(r0 + k * B, B)],
                              ssem[k]).wait()
    pltpu.make_async_copy(rows[0].at[pl.ds(0, rem)],
                          acc.at[pl.ds(r0 + nfull * B, rem)],
                          ssem[nfull]).wait()
    stage.wait()
    plsc.subcore_barrier()

    _seg_sum_pipeline(eidx, tab_sh, acc, src_v, dst_v, rows, srcx, dstx,
                      gsem, ssem, wid, NBUF1, PIPE1)
    plsc.subcore_barrier()

    pltpu.sync_copy(acc.at[pl.ds(r0, RPT)], out.at[c, pl.ds(r0, RPT)])


@functools.partial(
    pl.kernel,
    out_type=jax.ShapeDtypeStruct((NPAD, 128), jnp.float32),
    mesh=_MESH,
    scratch_types=[
        pltpu.VMEM((NB, B), jnp.int32),
        pltpu.VMEM((NB, B), jnp.int32),
        [pltpu.VMEM((B, 16), jnp.float32)] * NBUF,
        pltpu.VMEM((1, B), jnp.int32),
        pltpu.VMEM((1, B), jnp.int32),
        pltpu.VMEM((RPT, 32), jnp.float32),
        pltpu.VMEM((RPT, 32), jnp.float32),
        pltpu.VMEM((RPT, 16), jnp.float32),
        pltpu.VMEM((RPT, 16), jnp.float32),
        pltpu.VMEM((RPT, 16), jnp.float32),
        pltpu.VMEM((16,), jnp.float32),
        pltpu.VMEM_SHARED((NPAD, 16), jnp.float32),
        pltpu.VMEM_SHARED((NPAD, 16), jnp.float32),
        [pltpu.SemaphoreType.DMA] * NBUF,
        [pltpu.SemaphoreType.DMA] * NBUF,
    ],
    compiler_params=_SC_PARAMS,
)
def _sc_layer2(t1x, acc1, b1, eidx, fin, src_v, dst_v, rows, srcx, dstx,
               a0_v, a1_v, r1_v, h_v, di_v, b1_v, acc, tab_sh, gsem, ssem):
    c = lax.axis_index("c")
    s = lax.axis_index("s")
    wid = c * NS + s
    r0 = s * RPT

    d0 = pltpu.async_copy(acc1.at[0, pl.ds(r0, RPT)], a0_v, gsem[0])
    d1 = pltpu.async_copy(acc1.at[1, pl.ds(r0, RPT)], a1_v, gsem[1])
    d2 = pltpu.async_copy(t1x.at[pl.ds(r0, RPT), pl.ds(32, 16)], r1_v,
                          gsem[2])
    d3 = pltpu.async_copy(b1, b1_v, gsem[3])
    d0.wait()
    d1.wait()
    d2.wait()
    d3.wait()
    b1r = b1_v[...]

    def h_row(i):
        srow = a0_v[i, pl.ds(0, 16)] + a1_v[i, pl.ds(0, 16)]
        degv = a0_v[i, pl.ds(16, 16)] + a1_v[i, pl.ds(16, 16)]
        dinvv = 1.0 / jnp.maximum(degv, 1.0)
        hrow = jnp.maximum(srow * dinvv + b1r + r1_v[i, pl.ds(0, 16)], 0.0)
        h_v[i, :] = hrow
        di_v[i, :] = dinvv
        a0_v[i, pl.ds(0, 16)] = jnp.zeros((16,), jnp.float32)

    def h_body(k, _):
        h_row(2 * k)
        h_row(2 * k + 1)
        return 0

    lax.fori_loop(0, RPT // 2, h_body, 0)
    p0 = pltpu.async_copy(h_v, tab_sh.at[pl.ds(r0, RPT)], gsem[0])
    p1 = pltpu.async_copy(a0_v.at[:, pl.ds(0, 16)], acc.at[pl.ds(r0, RPT)],
                          gsem[1])

    @pl.when(c == 0)
    def _():
        pltpu.async_copy(h_v, fin.at[pl.ds(r0, RPT), pl.ds(32, 16)],
                         ssem[0])
        pltpu.async_copy(di_v, fin.at[pl.ds(r0, RPT), pl.ds(48, 16)],
                         ssem[1])
        pltpu.make_async_copy(h_v, fin.at[pl.ds(r0, RPT), pl.ds(32, 16)],
                              ssem[0]).wait()
        pltpu.make_async_copy(di_v, fin.at[pl.ds(r0, RPT), pl.ds(48, 16)],
                              ssem[1]).wait()

    p0.wait()
    p1.wait()
    plsc.subcore_barrier()

    _seg_sum_pipeline(eidx, tab_sh, acc, src_v, dst_v, rows, srcx, dstx,
                      gsem, ssem, wid, NBUF, PIPE)
    plsc.subcore_barrier()

    pltpu.sync_copy(acc.at[pl.ds(r0, RPT)],
                    fin.at[pl.ds(r0, RPT), pl.ds(c * 16, 16)])


def _k1_body(x_ref, w_ref, oh_ref, t1x_ref):
    pc = jnp.dot(x_ref[:, :], w_ref[:, :], preferred_element_type=jnp.float32)
    t1x_ref[0:N, 0:48] = pc + oh_ref[:, :]
    t1x_ref[N:NPAD, 0:48] = jnp.zeros((NPAD - N, 48), jnp.float32)


def _k3_body(fin_ref, w2_ref, b2_ref, out_ref):
    s2 = fin_ref[0:N, 0:16] + fin_ref[0:N, 16:32]
    h = fin_ref[0:N, 32:48]
    dinv = fin_ref[0:N, 48:49]
    cat = jnp.concatenate([s2 * dinv, h], axis=1)
    z = jnp.dot(cat, w2_ref[:, :], preferred_element_type=jnp.float32)
    z = z + b2_ref[:, :]
    m = jnp.max(z, axis=1, keepdims=True)
    e = jnp.exp(z - m)
    lse = jnp.log(jnp.sum(e, axis=1, keepdims=True))
    out_ref[:, :] = z - m - lse


def kernel(x, edge_index, Wl1, Wr1, b1, Wl2, Wr2, b2):
    eidx = edge_index.astype(jnp.int32).reshape(2, NBAT, B)

    w = jnp.concatenate(
        [Wl1.T, jnp.zeros((D_IN, 16), jnp.float32), Wr1.T], axis=1)
    oh = jnp.concatenate(
        [jnp.zeros((1, 16), jnp.float32), jnp.ones((1, 16), jnp.float32),
         jnp.zeros((1, 16), jnp.float32)], axis=1)
    t1x = pl.pallas_call(
        _k1_body,
        out_shape=jax.ShapeDtypeStruct((NPAD, 128), jnp.float32),
    )(x, w, oh)

    acc1 = _sc_layer1(t1x, eidx)
    fin = _sc_layer2(t1x, acc1, b1, eidx)

    w2 = jnp.concatenate([Wl2.T, Wr2.T], axis=0)
    out = pl.pallas_call(
        _k3_body,
        out_shape=jax.ShapeDtypeStruct((N, D_OUT), jnp.float32),
    )(fin, w2, b2.reshape(1, D_OUT))
    return out
